# baseline (speedup 1.0000x reference)
import functools

import numpy as np
import jax
import jax.numpy as jnp
from jax import lax
from jax.experimental import pallas as pl
from jax.experimental.pallas import tpu as pltpu
from jax.experimental.pallas import tpu_sc as plsc

SCALE = 0.0625
OUT = 7
C = 256
H = 50
W = 50
SMAX = 6
NT = SMAX * SMAX
NROI = 1000
NROI_PAD = 1024
NCELL = OUT * OUT
NPAIR = NCELL * NROI_PAD
HP = 56
NROWS = NT * HP * HP
NEG = -1e30
RECIP7 = float(np.float32(1.0) / np.float32(7.0))

SC_NC = 2
SC_NS = 16
SC_NW = SC_NC * SC_NS
BPW = NPAIR // SC_NW
SC_CHUNK = 112
CG = C // 2


def _k1_body(f_ref, b_ref, tab_ref, idx_ref, a_scr, w_scr, h0_scr, h1_scr,
             sem0, sem1):
    bx = b_ref[...] * SCALE
    bi = jnp.round(bx).astype(jnp.int32)
    x1, y1, x2, y2 = bi[0], bi[1], bi[2], bi[3]
    rw = jnp.maximum(x2 - x1 + 1, 1)
    rh = jnp.maximum(y2 - y1 + 1, 1)

    def _win(v1, r, p, hi):
        lo_f = jnp.floor((p * r).astype(jnp.float32) * RECIP7)
        hi_f = jnp.ceil(((p + 1) * r).astype(jnp.float32) * RECIP7)
        s = jnp.clip(v1 + lo_f.astype(jnp.int32), 0, hi - 1)
        e = jnp.clip(v1 + hi_f.astype(jnp.int32), 1, hi)
        e = jnp.maximum(e, s + 1)
        sz = jnp.clip(e - s, 1, SMAX)
        return s, sz

    for ph in range(OUT):
        hs, sh = _win(y1, rh, ph, H)
        for pw in range(OUT):
            ws, sw = _win(x1, rw, pw, W)
            t = (sw - 1) * SMAX + (sh - 1)
            idx_ref[ph * OUT + pw] = t * (HP * HP) + hs * HP + ws

    a_scr[...] = jnp.full((56, 56, C), NEG, jnp.bfloat16)
    a_scr[0:H, 0:W, :] = f_ref[...].astype(jnp.bfloat16)

    hbufs = (h0_scr, h1_scr)
    sems = (sem0, sem1)
    dmas = [None, None]
    g = 0
    for sw_ in range(1, SMAX + 1):
        if sw_ == 1:
            w_scr[...] = a_scr[...]
        else:
            w_scr[:, 0:51, :] = jnp.maximum(w_scr[:, 0:51, :],
                                            a_scr[:, sw_ - 1:sw_ + 50, :])
        for sh_ in range(1, SMAX + 1):
            hb = hbufs[g % 2]
            if dmas[g % 2] is not None:
                dmas[g % 2].wait()
            if sh_ == 1:
                hb[...] = w_scr[...]
            else:
                hprev = hbufs[(g - 1) % 2]
                hb[0:51, :, :] = jnp.maximum(hprev[0:51, :, :],
                                             w_scr[sh_ - 1:sh_ + 50, :, :])
            t = (sw_ - 1) * SMAX + (sh_ - 1)
            dma = pltpu.make_async_copy(hb, tab_ref.at[t], sems[g % 2])
            dma.start()
            dmas[g % 2] = dma
            g += 1
    dmas[0].wait()
    dmas[1].wait()


def _build_tables(f_hwc, boxes_r):
    return pl.pallas_call(
        _k1_body,
        out_shape=[
            jax.ShapeDtypeStruct((NT, HP, HP, C), jnp.bfloat16),
            jax.ShapeDtypeStruct((NCELL, 8, 128), jnp.int32),
        ],
        in_specs=[
            pl.BlockSpec(memory_space=pltpu.VMEM),
            pl.BlockSpec(memory_space=pltpu.VMEM),
        ],
        out_specs=[
            pl.BlockSpec(memory_space=pltpu.MemorySpace.HBM),
            pl.BlockSpec(memory_space=pltpu.VMEM),
        ],
        scratch_shapes=[
            pltpu.VMEM((56, 56, C), jnp.bfloat16),
            pltpu.VMEM((56, 56, C), jnp.bfloat16),
            pltpu.VMEM((56, 56, C), jnp.bfloat16),
            pltpu.VMEM((56, 56, C), jnp.bfloat16),
            pltpu.SemaphoreType.DMA,
            pltpu.SemaphoreType.DMA,
        ],
    )(f_hwc, boxes_r)


def _sc_body(tab_hbm, idx_hbm, out_hbm, idx0, idx1, rows0, rows1,
             gsem, ssem0, ssem1):
    wid = lax.axis_index("s") * SC_NC + lax.axis_index("c")
    base = wid * BPW
    idxb = (idx0, idx1)
    rowsb = (rows0, rows1)
    ssems = (ssem0, ssem1)
    nch = BPW // SC_CHUNK
    scat = [None, None]
    for j in range(nch):
        b = j % 2
        off = base + j * SC_CHUNK
        if scat[b] is not None:
            scat[b].wait()
        pltpu.sync_copy(idx_hbm.at[pl.ds(off, SC_CHUNK)], idxb[b])
        pltpu.async_copy(tab_hbm.at[idxb[b]], rowsb[b], gsem).wait()
        s = pltpu.async_copy(rowsb[b], out_hbm.at[pl.ds(off, SC_CHUNK)],
                             ssems[b])
        scat[b] = s
    scat[0].wait()
    scat[1].wait()


def _gather_rows(tab, idx):
    mesh = plsc.VectorSubcoreMesh(core_axis_name="c", subcore_axis_name="s",
                                  num_cores=SC_NC, num_subcores=SC_NS)
    fn = functools.partial(
        pl.kernel,
        mesh=mesh,
        out_type=jax.ShapeDtypeStruct((NPAIR, CG), jnp.float32),
        scratch_types=[
            pltpu.VMEM((SC_CHUNK,), jnp.int32),
            pltpu.VMEM((SC_CHUNK,), jnp.int32),
            pltpu.VMEM((SC_CHUNK, CG), jnp.float32),
            pltpu.VMEM((SC_CHUNK, CG), jnp.float32),
            pltpu.SemaphoreType.DMA,
            pltpu.SemaphoreType.DMA,
            pltpu.SemaphoreType.DMA,
        ],
    )(_sc_body)
    return fn(tab, idx)


def _k3_body(x_ref, w1_ref, w2_ref, wb_ref, wc_ref, b1_ref, b2_ref, bb_ref,
             bc_ref, pt_ref, pl_ref, acc):
    i = pl.program_id(0)
    x = x_ref[0]
    w = w1_ref[:, 0, 0, :].astype(jnp.bfloat16)
    prod = jnp.dot(x, w, preferred_element_type=jnp.float32)

    @pl.when(i == 0)
    def _():
        acc[...] = prod

    @pl.when(i > 0)
    def _():
        acc[...] += prod

    @pl.when(i == NCELL - 1)
    def _():
        h1 = jnp.maximum(acc[...] + b1_ref[...], 0.0).astype(jnp.bfloat16)
        h2 = jnp.dot(h1, w2_ref[...].astype(jnp.bfloat16),
                     preferred_element_type=jnp.float32) + b2_ref[...]
        h2 = jnp.maximum(h2, 0.0).astype(jnp.bfloat16)
        pt_ref[...] = jnp.dot(h2, wb_ref[...].astype(jnp.bfloat16),
                              preferred_element_type=jnp.float32) + bb_ref[...]
        pl_ref[...] = jnp.dot(h2, wc_ref[...].astype(jnp.bfloat16),
                              preferred_element_type=jnp.float32) + bc_ref[...]


def _mlp_head(xv, w1r, w2, wbox, wcls, b1, b2, bbox, bcls):
    rep = w2.shape[0]
    return pl.pallas_call(
        _k3_body,
        grid=(NCELL,),
        in_specs=[
            pl.BlockSpec((1, NROI_PAD, C), lambda i: (i, 0, 0)),
            pl.BlockSpec((C, 1, 1, rep), lambda i: (0, i, 0, 0)),
            pl.BlockSpec((rep, rep), lambda i: (0, 0)),
            pl.BlockSpec((rep, 4 * 21), lambda i: (0, 0)),
            pl.BlockSpec((rep, 21), lambda i: (0, 0)),
            pl.BlockSpec((1, rep), lambda i: (0, 0)),
            pl.BlockSpec((1, rep), lambda i: (0, 0)),
            pl.BlockSpec((1, 4 * 21), lambda i: (0, 0)),
            pl.BlockSpec((1, 21), lambda i: (0, 0)),
        ],
        out_specs=[
            pl.BlockSpec((NROI_PAD, 4 * 21), lambda i: (0, 0)),
            pl.BlockSpec((NROI_PAD, 21), lambda i: (0, 0)),
        ],
        out_shape=[
            jax.ShapeDtypeStruct((NROI_PAD, 4 * 21), jnp.float32),
            jax.ShapeDtypeStruct((NROI_PAD, 21), jnp.float32),
        ],
        scratch_shapes=[pltpu.VMEM((NROI_PAD, rep), jnp.float32)],
    )(xv, w1r, w2, wbox, wcls, b1, b2, bbox, bcls)


def kernel(backbone_features, proposals, gt_boxes, gt_classes,
           W1, b1, W2, b2, Wbox, bbox, Wcls, bcls):
    f_hwc = jnp.transpose(backbone_features[0], (1, 2, 0))
    boxes_t = jnp.zeros((4, NROI_PAD), jnp.float32)
    boxes_t = boxes_t.at[:, :NROI].set(proposals[0].T)
    boxes_r = boxes_t.reshape(4, 8, 128)

    tab, idx = _build_tables(f_hwc, boxes_r)
    tab_w = jax.lax.bitcast_convert_type(
        tab.reshape(NROWS, CG, 2), jnp.float32)
    x_w = _gather_rows(tab_w, idx.reshape(NPAIR))
    x = jax.lax.bitcast_convert_type(x_w, jnp.bfloat16)

    rep = W2.shape[0]
    out_t, out_l = _mlp_head(
        x.reshape(NCELL, NROI_PAD, C),
        W1.reshape(C, NCELL, 1, rep),
        W2, Wbox, Wcls,
        b1.reshape(1, rep), b2.reshape(1, rep),
        bbox.reshape(1, 4 * 21), bcls.reshape(1, 21),
    )
    return out_t[:NROI], out_l[:NROI]


# speedup vs baseline: 3.1100x; 3.1100x over previous
import functools

import numpy as np
import jax
import jax.numpy as jnp
from jax import lax
from jax.experimental import pallas as pl
from jax.experimental.pallas import tpu as pltpu
from jax.experimental.pallas import tpu_sc as plsc

SCALE = 0.0625
OUT = 7
C = 256
H = 50
W = 50
SMAX = 6
NT = SMAX * SMAX
NROI = 1000
NROI_PAD = 1024
NCELL = OUT * OUT
NPAIR = NCELL * NROI_PAD
HP = 56
NROWS = NT * HP * HP
NEG = -1e30
RECIP7 = float(np.float32(1.0) / np.float32(7.0))

SC_NC = 2
SC_NS = 16
SC_NW = SC_NC * SC_NS
BPW = NPAIR // SC_NW
SC_CHUNK = 112
CG = C // 2


def _k1_body(f_ref, b_ref, tab_ref, idx_ref, a_scr, w_scr, h0_scr, h1_scr,
             sem0, sem1):
    bx = b_ref[...] * SCALE
    bi = jnp.round(bx).astype(jnp.int32)
    x1, y1, x2, y2 = bi[0], bi[1], bi[2], bi[3]
    rw = jnp.maximum(x2 - x1 + 1, 1)
    rh = jnp.maximum(y2 - y1 + 1, 1)

    def _win(v1, r, p, hi):
        lo_f = jnp.floor((p * r).astype(jnp.float32) * RECIP7)
        hi_f = jnp.ceil(((p + 1) * r).astype(jnp.float32) * RECIP7)
        s = jnp.clip(v1 + lo_f.astype(jnp.int32), 0, hi - 1)
        e = jnp.clip(v1 + hi_f.astype(jnp.int32), 1, hi)
        e = jnp.maximum(e, s + 1)
        sz = jnp.clip(e - s, 1, SMAX)
        return s, sz

    for ph in range(OUT):
        hs, sh = _win(y1, rh, ph, H)
        for pw in range(OUT):
            ws, sw = _win(x1, rw, pw, W)
            t = (sw - 1) * SMAX + (sh - 1)
            idx_ref[ph * OUT + pw] = t * (HP * HP) + hs * HP + ws

    a_scr[...] = jnp.full((56, 56, C), NEG, jnp.float32)
    a_scr[0:H, 0:W, :] = f_ref[...]

    hbufs = (h0_scr, h1_scr)
    sems = (sem0, sem1)
    dmas = [None, None]
    g = 0
    for sw_ in range(1, SMAX + 1):
        if sw_ == 1:
            w_scr[...] = a_scr[...]
        else:
            w_scr[:, 0:51, :] = jnp.maximum(w_scr[:, 0:51, :],
                                            a_scr[:, sw_ - 1:sw_ + 50, :])
        for sh_ in range(1, SMAX + 1):
            hb = hbufs[g % 2]
            if dmas[g % 2] is not None:
                dmas[g % 2].wait()
            if sh_ == 1:
                hb[...] = w_scr[...]
            else:
                hprev = hbufs[(g - 1) % 2]
                hb[0:51, :, :] = jnp.maximum(hprev[0:51, :, :],
                                             w_scr[sh_ - 1:sh_ + 50, :, :])
            t = (sw_ - 1) * SMAX + (sh_ - 1)
            dma = pltpu.make_async_copy(hb, tab_ref.at[t], sems[g % 2])
            dma.start()
            dmas[g % 2] = dma
            g += 1
    dmas[0].wait()
    dmas[1].wait()


def _build_tables(f_hwc, boxes_r):
    return pl.pallas_call(
        _k1_body,
        out_shape=[
            jax.ShapeDtypeStruct((NT, HP, HP, C), jnp.float32),
            jax.ShapeDtypeStruct((NCELL, 8, 128), jnp.int32),
        ],
        in_specs=[
            pl.BlockSpec(memory_space=pltpu.VMEM),
            pl.BlockSpec(memory_space=pltpu.VMEM),
        ],
        out_specs=[
            pl.BlockSpec(memory_space=pltpu.MemorySpace.HBM),
            pl.BlockSpec(memory_space=pltpu.VMEM),
        ],
        scratch_shapes=[
            pltpu.VMEM((56, 56, C), jnp.float32),
            pltpu.VMEM((56, 56, C), jnp.float32),
            pltpu.VMEM((56, 56, C), jnp.float32),
            pltpu.VMEM((56, 56, C), jnp.float32),
            pltpu.SemaphoreType.DMA,
            pltpu.SemaphoreType.DMA,
        ],
    )(f_hwc, boxes_r)


NCH = BPW // SC_CHUNK


def _sc_body(tab_hbm, idx_hbm, out_hbm, idx_all, rows0, rows1,
             gsem0, gsem1, ssem0, ssem1):
    wid = lax.axis_index("s") * SC_NC + lax.axis_index("c")
    base = wid * BPW
    pltpu.sync_copy(idx_hbm.at[wid], idx_all)
    rowsb = (rows0, rows1)
    gsems = (gsem0, gsem1)
    ssems = (ssem0, ssem1)
    gat = [None, None]
    scat = [None, None]
    for j in range(NCH):
        b = j % 2
        if scat[b] is not None:
            scat[b].wait()
        gat[b] = pltpu.async_copy(tab_hbm.at[idx_all.at[j]], rowsb[b],
                                  gsems[b])
        if j >= 1:
            pb = (j - 1) % 2
            gat[pb].wait()
            scat[pb] = pltpu.async_copy(
                rowsb[pb],
                out_hbm.at[pl.ds(base + (j - 1) * SC_CHUNK, SC_CHUNK)],
                ssems[pb])
    lb = (NCH - 1) % 2
    gat[lb].wait()
    scat[lb] = pltpu.async_copy(
        rowsb[lb],
        out_hbm.at[pl.ds(base + (NCH - 1) * SC_CHUNK, SC_CHUNK)],
        ssems[lb])
    scat[0].wait()
    scat[1].wait()


def _gather_rows(tab, idx):
    mesh = plsc.VectorSubcoreMesh(core_axis_name="c", subcore_axis_name="s",
                                  num_cores=SC_NC, num_subcores=SC_NS)
    fn = functools.partial(
        pl.kernel,
        mesh=mesh,
        out_type=jax.ShapeDtypeStruct((NPAIR, C), jnp.float32),
        scratch_types=[
            pltpu.VMEM((NCH, SC_CHUNK), jnp.int32),
            pltpu.VMEM((SC_CHUNK, C), jnp.float32),
            pltpu.VMEM((SC_CHUNK, C), jnp.float32),
            pltpu.SemaphoreType.DMA,
            pltpu.SemaphoreType.DMA,
            pltpu.SemaphoreType.DMA,
            pltpu.SemaphoreType.DMA,
        ],
    )(_sc_body)
    return fn(tab, idx)


def _k3_body(x_ref, w1_ref, w2_ref, wb_ref, wc_ref, b1_ref, b2_ref, bb_ref,
             bc_ref, pt_ref, pl_ref, acc):
    i = pl.program_id(0)
    x = x_ref[0].astype(jnp.bfloat16)
    w = w1_ref[:, 0, 0, :].astype(jnp.bfloat16)
    prod = jnp.dot(x, w, preferred_element_type=jnp.float32)

    @pl.when(i == 0)
    def _():
        acc[...] = prod

    @pl.when(i > 0)
    def _():
        acc[...] += prod

    @pl.when(i == NCELL - 1)
    def _():
        h1 = jnp.maximum(acc[...] + b1_ref[...], 0.0).astype(jnp.bfloat16)
        h2 = jnp.dot(h1, w2_ref[...].astype(jnp.bfloat16),
                     preferred_element_type=jnp.float32) + b2_ref[...]
        h2 = jnp.maximum(h2, 0.0).astype(jnp.bfloat16)
        pt_ref[...] = jnp.dot(h2, wb_ref[...].astype(jnp.bfloat16),
                              preferred_element_type=jnp.float32) + bb_ref[...]
        pl_ref[...] = jnp.dot(h2, wc_ref[...].astype(jnp.bfloat16),
                              preferred_element_type=jnp.float32) + bc_ref[...]


def _mlp_head(xv, w1r, w2, wbox, wcls, b1, b2, bbox, bcls):
    rep = w2.shape[0]
    return pl.pallas_call(
        _k3_body,
        grid=(NCELL,),
        in_specs=[
            pl.BlockSpec((1, NROI_PAD, C), lambda i: (i, 0, 0)),
            pl.BlockSpec((C, 1, 1, rep), lambda i: (0, i, 0, 0)),
            pl.BlockSpec((rep, rep), lambda i: (0, 0)),
            pl.BlockSpec((rep, 4 * 21), lambda i: (0, 0)),
            pl.BlockSpec((rep, 21), lambda i: (0, 0)),
            pl.BlockSpec((1, rep), lambda i: (0, 0)),
            pl.BlockSpec((1, rep), lambda i: (0, 0)),
            pl.BlockSpec((1, 4 * 21), lambda i: (0, 0)),
            pl.BlockSpec((1, 21), lambda i: (0, 0)),
        ],
        out_specs=[
            pl.BlockSpec((NROI_PAD, 4 * 21), lambda i: (0, 0)),
            pl.BlockSpec((NROI_PAD, 21), lambda i: (0, 0)),
        ],
        out_shape=[
            jax.ShapeDtypeStruct((NROI_PAD, 4 * 21), jnp.float32),
            jax.ShapeDtypeStruct((NROI_PAD, 21), jnp.float32),
        ],
        scratch_shapes=[pltpu.VMEM((NROI_PAD, rep), jnp.float32)],
    )(xv, w1r, w2, wbox, wcls, b1, b2, bbox, bcls)


def kernel(backbone_features, proposals, gt_boxes, gt_classes,
           W1, b1, W2, b2, Wbox, bbox, Wcls, bcls):
    f_hwc = jnp.transpose(backbone_features[0], (1, 2, 0))
    boxes_t = jnp.zeros((4, NROI_PAD), jnp.float32)
    boxes_t = boxes_t.at[:, :NROI].set(proposals[0].T)
    boxes_r = boxes_t.reshape(4, 8, 128)

    tab, idx = _build_tables(f_hwc, boxes_r)
    x = _gather_rows(tab.reshape(NROWS, C),
                     idx.reshape(NPAIR).reshape(SC_NW, NCH, SC_CHUNK))

    rep = W2.shape[0]
    out_t, out_l = _mlp_head(
        x.reshape(NCELL, NROI_PAD, C),
        W1.reshape(C, NCELL, 1, rep),
        W2, Wbox, Wcls,
        b1.reshape(1, rep), b2.reshape(1, rep),
        bbox.reshape(1, 4 * 21), bcls.reshape(1, 21),
    )
    return out_t[:NROI], out_l[:NROI]


# speedup vs baseline: 3.7423x; 1.2033x over previous
import functools

import numpy as np
import jax
import jax.numpy as jnp
from jax import lax
from jax.experimental import pallas as pl
from jax.experimental.pallas import tpu as pltpu
from jax.experimental.pallas import tpu_sc as plsc

SCALE = 0.0625
OUT = 7
C = 256
H = 50
W = 50
SMAX = 6
NT = SMAX * SMAX
NROI = 1000
NROI_PAD = 1024
NCELL = OUT * OUT
NPAIR = NCELL * NROI_PAD
HP = 56
NROWS = NT * HP * HP
NEG = -1e30
RECIP7 = float(np.float32(1.0) / np.float32(7.0))

SC_NC = 2
SC_NS = 16
SC_NW = SC_NC * SC_NS
BPW = NPAIR // SC_NW
SC_CHUNK = 112
CG = C // 2


def _k1_body(f_ref, b_ref, tab_ref, idx_ref, a_scr, w_scr, h0_scr, h1_scr,
             sem0, sem1):
    bx = b_ref[...] * SCALE
    bi = jnp.round(bx).astype(jnp.int32)
    x1, y1, x2, y2 = bi[0], bi[1], bi[2], bi[3]
    rw = jnp.maximum(x2 - x1 + 1, 1)
    rh = jnp.maximum(y2 - y1 + 1, 1)

    def _win(v1, r, p, hi):
        lo_f = jnp.floor((p * r).astype(jnp.float32) * RECIP7)
        hi_f = jnp.ceil(((p + 1) * r).astype(jnp.float32) * RECIP7)
        s = jnp.clip(v1 + lo_f.astype(jnp.int32), 0, hi - 1)
        e = jnp.clip(v1 + hi_f.astype(jnp.int32), 1, hi)
        e = jnp.maximum(e, s + 1)
        sz = jnp.clip(e - s, 1, SMAX)
        return s, sz

    for ph in range(OUT):
        hs, sh = _win(y1, rh, ph, H)
        for pw in range(OUT):
            ws, sw = _win(x1, rw, pw, W)
            t = (sw - 1) * SMAX + (sh - 1)
            idx_ref[ph * OUT + pw] = t * (HP * HP) + hs * HP + ws

    a_scr[...] = jnp.full((56, 56, C), NEG, jnp.float32)
    a_scr[0:H, 0:W, :] = f_ref[...]

    hbufs = (h0_scr, h1_scr)
    sems = (sem0, sem1)
    dmas = [None, None]
    g = 0
    for sw_ in range(1, SMAX + 1):
        if sw_ == 1:
            w_scr[...] = a_scr[...]
        else:
            w_scr[:, 0:51, :] = jnp.maximum(w_scr[:, 0:51, :],
                                            a_scr[:, sw_ - 1:sw_ + 50, :])
        for sh_ in range(1, SMAX + 1):
            hb = hbufs[g % 2]
            if dmas[g % 2] is not None:
                dmas[g % 2].wait()
            if sh_ == 1:
                hb[...] = w_scr[...]
            else:
                hprev = hbufs[(g - 1) % 2]
                hb[0:51, :, :] = jnp.maximum(hprev[0:51, :, :],
                                             w_scr[sh_ - 1:sh_ + 50, :, :])
            t = (sw_ - 1) * SMAX + (sh_ - 1)
            dma = pltpu.make_async_copy(hb, tab_ref.at[t], sems[g % 2])
            dma.start()
            dmas[g % 2] = dma
            g += 1
    dmas[0].wait()
    dmas[1].wait()


def _build_tables(f_hwc, boxes_r):
    return pl.pallas_call(
        _k1_body,
        out_shape=[
            jax.ShapeDtypeStruct((NT, HP, HP, C), jnp.float32),
            jax.ShapeDtypeStruct((NCELL, 8, 128), jnp.int32),
        ],
        in_specs=[
            pl.BlockSpec(memory_space=pltpu.VMEM),
            pl.BlockSpec(memory_space=pltpu.VMEM),
        ],
        out_specs=[
            pl.BlockSpec(memory_space=pltpu.MemorySpace.HBM),
            pl.BlockSpec(memory_space=pltpu.VMEM),
        ],
        scratch_shapes=[
            pltpu.VMEM((56, 56, C), jnp.float32),
            pltpu.VMEM((56, 56, C), jnp.float32),
            pltpu.VMEM((56, 56, C), jnp.float32),
            pltpu.VMEM((56, 56, C), jnp.float32),
            pltpu.SemaphoreType.DMA,
            pltpu.SemaphoreType.DMA,
        ],
    )(f_hwc, boxes_r)


NCH = BPW // SC_CHUNK


def _sc_body(tab_hbm, idx_hbm, out_hbm, idx_all, rows0, rows1,
             gsem0, gsem1, ssem0, ssem1):
    wid = lax.axis_index("s") * SC_NC + lax.axis_index("c")
    base = wid * BPW
    pltpu.sync_copy(idx_hbm.at[wid], idx_all)
    rowsb = (rows0, rows1)
    gsems = (gsem0, gsem1)
    ssems = (ssem0, ssem1)
    gat = [None, None]
    scat = [None, None]
    for j in range(NCH):
        b = j % 2
        if scat[b] is not None:
            scat[b].wait()
        gat[b] = pltpu.async_copy(tab_hbm.at[idx_all.at[j]], rowsb[b],
                                  gsems[b])
        if j >= 1:
            pb = (j - 1) % 2
            gat[pb].wait()
            scat[pb] = pltpu.async_copy(
                rowsb[pb],
                out_hbm.at[pl.ds(base + (j - 1) * SC_CHUNK, SC_CHUNK)],
                ssems[pb])
    lb = (NCH - 1) % 2
    gat[lb].wait()
    scat[lb] = pltpu.async_copy(
        rowsb[lb],
        out_hbm.at[pl.ds(base + (NCH - 1) * SC_CHUNK, SC_CHUNK)],
        ssems[lb])
    scat[0].wait()
    scat[1].wait()


def _gather_rows(tab, idx):
    mesh = plsc.VectorSubcoreMesh(core_axis_name="c", subcore_axis_name="s",
                                  num_cores=SC_NC, num_subcores=SC_NS)
    fn = functools.partial(
        pl.kernel,
        mesh=mesh,
        out_type=jax.ShapeDtypeStruct((NPAIR, C), jnp.float32),
        scratch_types=[
            pltpu.VMEM((NCH, SC_CHUNK), jnp.int32),
            pltpu.VMEM((SC_CHUNK, C), jnp.float32),
            pltpu.VMEM((SC_CHUNK, C), jnp.float32),
            pltpu.SemaphoreType.DMA,
            pltpu.SemaphoreType.DMA,
            pltpu.SemaphoreType.DMA,
            pltpu.SemaphoreType.DMA,
        ],
    )(_sc_body)
    return fn(tab, idx)


def _k3_body(x_ref, w1_ref, w2_ref, wb_ref, wc_ref, b1_ref, b2_ref, bb_ref,
             bc_ref, pt_ref, pl_ref, acc):
    i = pl.program_id(0)
    x = x_ref[0].astype(jnp.bfloat16)
    w = w1_ref[0].astype(jnp.bfloat16)
    prod = jnp.dot(x, w, preferred_element_type=jnp.float32)

    @pl.when(i == 0)
    def _():
        acc[...] = prod

    @pl.when(i > 0)
    def _():
        acc[...] += prod

    @pl.when(i == NCELL - 1)
    def _():
        h1 = jnp.maximum(acc[...] + b1_ref[...], 0.0).astype(jnp.bfloat16)
        h2 = jnp.dot(h1, w2_ref[...].astype(jnp.bfloat16),
                     preferred_element_type=jnp.float32) + b2_ref[...]
        h2 = jnp.maximum(h2, 0.0).astype(jnp.bfloat16)
        pt_ref[...] = jnp.dot(h2, wb_ref[...].astype(jnp.bfloat16),
                              preferred_element_type=jnp.float32) + bb_ref[...]
        pl_ref[...] = jnp.dot(h2, wc_ref[...].astype(jnp.bfloat16),
                              preferred_element_type=jnp.float32) + bc_ref[...]


def _mlp_head(xv, w1r, w2, wbox, wcls, b1, b2, bbox, bcls):
    rep = w2.shape[0]
    return pl.pallas_call(
        _k3_body,
        grid=(NCELL,),
        in_specs=[
            pl.BlockSpec((1, NROI_PAD, C), lambda i: (i, 0, 0)),
            pl.BlockSpec((1, C, rep), lambda i: (i, 0, 0)),
            pl.BlockSpec((rep, rep), lambda i: (0, 0)),
            pl.BlockSpec((rep, 4 * 21), lambda i: (0, 0)),
            pl.BlockSpec((rep, 21), lambda i: (0, 0)),
            pl.BlockSpec((1, rep), lambda i: (0, 0)),
            pl.BlockSpec((1, rep), lambda i: (0, 0)),
            pl.BlockSpec((1, 4 * 21), lambda i: (0, 0)),
            pl.BlockSpec((1, 21), lambda i: (0, 0)),
        ],
        out_specs=[
            pl.BlockSpec((NROI_PAD, 4 * 21), lambda i: (0, 0)),
            pl.BlockSpec((NROI_PAD, 21), lambda i: (0, 0)),
        ],
        out_shape=[
            jax.ShapeDtypeStruct((NROI_PAD, 4 * 21), jnp.float32),
            jax.ShapeDtypeStruct((NROI_PAD, 21), jnp.float32),
        ],
        scratch_shapes=[pltpu.VMEM((NROI_PAD, rep), jnp.float32)],
    )(xv, w1r, w2, wbox, wcls, b1, b2, bbox, bcls)


def kernel(backbone_features, proposals, gt_boxes, gt_classes,
           W1, b1, W2, b2, Wbox, bbox, Wcls, bcls):
    f_hwc = jnp.transpose(backbone_features[0], (1, 2, 0))
    boxes_t = jnp.zeros((4, NROI_PAD), jnp.float32)
    boxes_t = boxes_t.at[:, :NROI].set(proposals[0].T)
    boxes_r = boxes_t.reshape(4, 8, 128)

    tab, idx = _build_tables(f_hwc, boxes_r)
    x = _gather_rows(tab.reshape(NROWS, C),
                     idx.reshape(NPAIR).reshape(SC_NW, NCH, SC_CHUNK))

    rep = W2.shape[0]
    out_t, out_l = _mlp_head(
        x.reshape(NCELL, NROI_PAD, C),
        jnp.transpose(W1.reshape(C, NCELL, rep), (1, 0, 2)),
        W2, Wbox, Wcls,
        b1.reshape(1, rep), b2.reshape(1, rep),
        bbox.reshape(1, 4 * 21), bcls.reshape(1, 21),
    )
    return out_t[:NROI], out_l[:NROI]


# speedup vs baseline: 4.7989x; 1.2823x over previous
import functools

import numpy as np
import jax
import jax.numpy as jnp
from jax import lax
from jax.experimental import pallas as pl
from jax.experimental.pallas import tpu as pltpu
from jax.experimental.pallas import tpu_sc as plsc

SCALE = 0.0625
OUT = 7
C = 256
H = 50
W = 50
SMAX = 6
NT = SMAX * SMAX
NROI = 1000
NROI_PAD = 1024
NCELL = OUT * OUT
NPAIR = NCELL * NROI_PAD
HP = 56
NROWS = NT * HP * HP
NEG = -1e30
RECIP7 = float(np.float32(1.0) / np.float32(7.0))

SC_NC = 2
SC_NS = 16
SC_NW = SC_NC * SC_NS
BPW = NPAIR // SC_NW
SC_CHUNK = 112
CG = C // 2


def _k1_body(f_ref, b_ref, tab_ref, idx_ref, a_scr, w_scr, h0_scr, h1_scr,
             sem0, sem1):
    bx = b_ref[...] * SCALE
    bi = jnp.round(bx).astype(jnp.int32)
    x1, y1, x2, y2 = bi[0], bi[1], bi[2], bi[3]
    rw = jnp.maximum(x2 - x1 + 1, 1)
    rh = jnp.maximum(y2 - y1 + 1, 1)

    def _win(v1, r, p, hi):
        lo_f = jnp.floor((p * r).astype(jnp.float32) * RECIP7)
        hi_f = jnp.ceil(((p + 1) * r).astype(jnp.float32) * RECIP7)
        s = jnp.clip(v1 + lo_f.astype(jnp.int32), 0, hi - 1)
        e = jnp.clip(v1 + hi_f.astype(jnp.int32), 1, hi)
        e = jnp.maximum(e, s + 1)
        sz = jnp.clip(e - s, 1, SMAX)
        return s, sz

    for ph in range(OUT):
        hs, sh = _win(y1, rh, ph, H)
        for pw in range(OUT):
            ws, sw = _win(x1, rw, pw, W)
            t = (sw - 1) * SMAX + (sh - 1)
            idx_ref[ph * OUT + pw] = t * (HP * HP) + hs * HP + ws

    a_scr[...] = jnp.full((56, 56, C), NEG, jnp.float32)
    a_scr[0:H, 0:W, :] = f_ref[...]

    hbufs = (h0_scr, h1_scr)
    sems = (sem0, sem1)
    dmas = [None, None]
    g = 0
    for sw_ in range(1, SMAX + 1):
        if sw_ == 1:
            w_scr[...] = a_scr[...]
        else:
            w_scr[:, 0:51, :] = jnp.maximum(w_scr[:, 0:51, :],
                                            a_scr[:, sw_ - 1:sw_ + 50, :])
        for sh_ in range(1, SMAX + 1):
            hb = hbufs[g % 2]
            if dmas[g % 2] is not None:
                dmas[g % 2].wait()
            if sh_ == 1:
                hb[...] = w_scr[...]
            else:
                hprev = hbufs[(g - 1) % 2]
                hb[0:51, :, :] = jnp.maximum(hprev[0:51, :, :],
                                             w_scr[sh_ - 1:sh_ + 50, :, :])
            t = (sw_ - 1) * SMAX + (sh_ - 1)
            dma = pltpu.make_async_copy(hb, tab_ref.at[t], sems[g % 2])
            dma.start()
            dmas[g % 2] = dma
            g += 1
    dmas[0].wait()
    dmas[1].wait()


def _build_tables(f_hwc, boxes_r):
    return pl.pallas_call(
        _k1_body,
        out_shape=[
            jax.ShapeDtypeStruct((NT, HP, HP, C), jnp.float32),
            jax.ShapeDtypeStruct((NCELL, 8, 128), jnp.int32),
        ],
        in_specs=[
            pl.BlockSpec(memory_space=pltpu.VMEM),
            pl.BlockSpec(memory_space=pltpu.VMEM),
        ],
        out_specs=[
            pl.BlockSpec(memory_space=pltpu.MemorySpace.HBM),
            pl.BlockSpec(memory_space=pltpu.VMEM),
        ],
        scratch_shapes=[
            pltpu.VMEM((56, 56, C), jnp.float32),
            pltpu.VMEM((56, 56, C), jnp.float32),
            pltpu.VMEM((56, 56, C), jnp.float32),
            pltpu.VMEM((56, 56, C), jnp.float32),
            pltpu.SemaphoreType.DMA,
            pltpu.SemaphoreType.DMA,
        ],
    )(f_hwc, boxes_r)


NCH = BPW // SC_CHUNK
NBUF = 4


def _sc_body(tab_hbm, idx_hbm, out_hbm, idx_all, rows0, rows1, rows2, rows3,
             gsem0, gsem1, gsem2, gsem3, ssem0, ssem1, ssem2, ssem3):
    wid = lax.axis_index("s") * SC_NC + lax.axis_index("c")
    base = wid * BPW
    pltpu.sync_copy(idx_hbm.at[wid], idx_all)
    rowsb = (rows0, rows1, rows2, rows3)
    gsems = (gsem0, gsem1, gsem2, gsem3)
    ssems = (ssem0, ssem1, ssem2, ssem3)
    gat = [None] * NBUF
    scat = [None] * NBUF
    for j in range(NCH):
        b = j % NBUF
        if scat[b] is not None:
            scat[b].wait()
        gat[b] = pltpu.async_copy(tab_hbm.at[idx_all.at[j]], rowsb[b],
                                  gsems[b])
        if j >= 1:
            pb = (j - 1) % NBUF
            gat[pb].wait()
            scat[pb] = pltpu.async_copy(
                rowsb[pb],
                out_hbm.at[pl.ds(base + (j - 1) * SC_CHUNK, SC_CHUNK)],
                ssems[pb])
    lb = (NCH - 1) % NBUF
    gat[lb].wait()
    scat[lb] = pltpu.async_copy(
        rowsb[lb],
        out_hbm.at[pl.ds(base + (NCH - 1) * SC_CHUNK, SC_CHUNK)],
        ssems[lb])
    for s in scat:
        if s is not None:
            s.wait()


def _gather_rows(tab, idx):
    mesh = plsc.VectorSubcoreMesh(core_axis_name="c", subcore_axis_name="s",
                                  num_cores=SC_NC, num_subcores=SC_NS)
    fn = functools.partial(
        pl.kernel,
        mesh=mesh,
        out_type=jax.ShapeDtypeStruct((NPAIR, C), jnp.float32),
        scratch_types=[
            pltpu.VMEM((NCH, SC_CHUNK), jnp.int32),
            pltpu.VMEM((SC_CHUNK, C), jnp.float32),
            pltpu.VMEM((SC_CHUNK, C), jnp.float32),
            pltpu.VMEM((SC_CHUNK, C), jnp.float32),
            pltpu.VMEM((SC_CHUNK, C), jnp.float32),
        ] + [pltpu.SemaphoreType.DMA] * 8,
    )(_sc_body)
    return fn(tab, idx)


def _w1t_body(w_ref, idx_ref, o_ref):
    for c in range(32):
        o_ref[:, c, :] = w_ref[c * NCELL:(c + 1) * NCELL, :]


def _repack_w1(w1, idx):
    rep = w1.shape[1]
    return pl.pallas_call(
        _w1t_body,
        grid=(C // 32,),
        in_specs=[
            pl.BlockSpec((32 * NCELL, rep), lambda i: (i, 0)),
            pl.BlockSpec(memory_space=pltpu.MemorySpace.HBM),
        ],
        out_specs=pl.BlockSpec((NCELL, 32, rep), lambda i: (0, i, 0)),
        out_shape=jax.ShapeDtypeStruct((NCELL, C, rep), jnp.float32),
    )(w1, idx)


def _k3_body(x_ref, w1_ref, w2_ref, wb_ref, wc_ref, b1_ref, b2_ref, bb_ref,
             bc_ref, pt_ref, pl_ref, acc):
    i = pl.program_id(0)
    prod = None
    for t in range(7):
        xt = x_ref[t].astype(jnp.bfloat16)
        wt = w1_ref[t].astype(jnp.bfloat16)
        d = jnp.dot(xt, wt, preferred_element_type=jnp.float32)
        prod = d if prod is None else prod + d

    @pl.when(i == 0)
    def _():
        acc[...] = prod

    @pl.when(i > 0)
    def _():
        acc[...] += prod

    @pl.when(i == 6)
    def _():
        h1 = jnp.maximum(acc[...] + b1_ref[...], 0.0).astype(jnp.bfloat16)
        h2 = jnp.dot(h1, w2_ref[...].astype(jnp.bfloat16),
                     preferred_element_type=jnp.float32) + b2_ref[...]
        h2 = jnp.maximum(h2, 0.0).astype(jnp.bfloat16)
        pt_ref[...] = jnp.dot(h2, wb_ref[...].astype(jnp.bfloat16),
                              preferred_element_type=jnp.float32) + bb_ref[...]
        pl_ref[...] = jnp.dot(h2, wc_ref[...].astype(jnp.bfloat16),
                              preferred_element_type=jnp.float32) + bc_ref[...]


def _mlp_head(xv, w1r, w2, wbox, wcls, b1, b2, bbox, bcls):
    rep = w2.shape[0]
    return pl.pallas_call(
        _k3_body,
        grid=(NCELL // 7,),
        in_specs=[
            pl.BlockSpec((7, NROI_PAD, C), lambda i: (i, 0, 0)),
            pl.BlockSpec((7, C, rep), lambda i: (i, 0, 0)),
            pl.BlockSpec((rep, rep), lambda i: (0, 0)),
            pl.BlockSpec((rep, 4 * 21), lambda i: (0, 0)),
            pl.BlockSpec((rep, 21), lambda i: (0, 0)),
            pl.BlockSpec((1, rep), lambda i: (0, 0)),
            pl.BlockSpec((1, rep), lambda i: (0, 0)),
            pl.BlockSpec((1, 4 * 21), lambda i: (0, 0)),
            pl.BlockSpec((1, 21), lambda i: (0, 0)),
        ],
        out_specs=[
            pl.BlockSpec((NROI_PAD, 4 * 21), lambda i: (0, 0)),
            pl.BlockSpec((NROI_PAD, 21), lambda i: (0, 0)),
        ],
        out_shape=[
            jax.ShapeDtypeStruct((NROI_PAD, 4 * 21), jnp.float32),
            jax.ShapeDtypeStruct((NROI_PAD, 21), jnp.float32),
        ],
        scratch_shapes=[pltpu.VMEM((NROI_PAD, rep), jnp.float32)],
    )(xv, w1r, w2, wbox, wcls, b1, b2, bbox, bcls)


def kernel(backbone_features, proposals, gt_boxes, gt_classes,
           W1, b1, W2, b2, Wbox, bbox, Wcls, bcls):
    f_hwc = jnp.transpose(backbone_features[0], (1, 2, 0))
    boxes_t = jnp.zeros((4, NROI_PAD), jnp.float32)
    boxes_t = boxes_t.at[:, :NROI].set(proposals[0].T)
    boxes_r = boxes_t.reshape(4, 8, 128)

    tab, idx = _build_tables(f_hwc, boxes_r)
    x = _gather_rows(tab.reshape(NROWS, C),
                     idx.reshape(NPAIR).reshape(SC_NW, NCH, SC_CHUNK))

    rep = W2.shape[0]
    w1t = _repack_w1(W1, idx)
    out_t, out_l = _mlp_head(
        x.reshape(NCELL, NROI_PAD, C),
        w1t,
        W2, Wbox, Wcls,
        b1.reshape(1, rep), b2.reshape(1, rep),
        bbox.reshape(1, 4 * 21), bcls.reshape(1, 21),
    )
    return out_t[:NROI], out_l[:NROI]
